# baseline (speedup 1.0000x reference)
import functools

import jax
import jax.numpy as jnp
import numpy as np
from jax import lax
from jax.experimental import pallas as pl
from jax.experimental.pallas import tpu as pltpu
from jax.experimental.pallas import tpu_sc as plsc

_INV_R2 = float(np.float32(1.0) / (np.float32(0.1) * np.float32(0.1)))

_NC, _NS, _L = 2, 16, 16
_NW = _NC * _NS
_CH = 196
_GB = 112


def _perm(v, idxvec):
    dn = lax.GatherDimensionNumbers(
        offset_dims=(), collapsed_slice_dims=(0,), start_index_map=(0,))
    return lax.gather(v, idxvec[:, None], dn, (1,),
                      mode=lax.GatherScatterMode.PROMISE_IN_BOUNDS)


def _bcast(v, lane):
    return _perm(v, jnp.full((_L,), lane, jnp.int32))


@functools.partial(jax.jit, static_argnames=("n_pix", "k_frag", "n_chan"))
def _render(idx_f, d_f, features, *, n_pix, k_frag, n_chan):
    K, C = k_frag, n_chan
    ppt = n_pix // _NW
    nch = ppt // _CH
    frag = _CH * K
    nstr = frag // _GB
    assert ppt % _CH == 0 and nch % 2 == 0 and frag % _GB == 0

    mesh = plsc.VectorSubcoreMesh(
        core_axis_name="c", subcore_axis_name="s",
        num_cores=_NC, num_subcores=_NS)

    @functools.partial(
        pl.kernel,
        out_type=jax.ShapeDtypeStruct((n_pix * C,), jnp.float32),
        mesh=mesh,
        compiler_params=pltpu.CompilerParams(
            needs_layout_passes=False, use_tc_tiling_on_sc=False),
        scratch_types=[
            pltpu.VMEM((frag,), jnp.int32),
            pltpu.VMEM((frag,), jnp.int32),
            pltpu.VMEM((frag + _L,), jnp.float32),
            pltpu.VMEM((frag + _L,), jnp.float32),
            pltpu.VMEM((frag, C), jnp.float32),
            pltpu.VMEM((frag, C), jnp.float32),
            pltpu.VMEM((_CH * C,), jnp.float32),
            pltpu.SemaphoreType.DMA,
            pltpu.SemaphoreType.DMA,
            pltpu.SemaphoreType.DMA,
            pltpu.SemaphoreType.DMA,
        ],
    )
    def k(idx_hbm, d_hbm, feat_hbm, out_hbm,
          idx_v0, idx_v1, d_v0, d_v1, rows_v0, rows_v1, out_v,
          sem_i0, sem_i1, sem_g0, sem_g1):
        idx_v = (idx_v0, idx_v1)
        d_v = (d_v0, d_v1)
        rows_v = (rows_v0, rows_v1)
        sem_i = (sem_i0, sem_i1)
        sem_g = (sem_g0, sem_g1)

        wid = lax.axis_index("s") * _NC + lax.axis_index("c")
        pix_base = wid * ppt
        frag_base = pix_base * K
        iota = lax.iota(jnp.int32, _L)
        ix1 = iota ^ 1
        ix2 = iota ^ 2
        ix4 = iota ^ 4

        def issue_in(ci, b):
            fb = frag_base + ci * frag
            pltpu.async_copy(idx_hbm.at[pl.ds(fb, frag)], idx_v[b], sem_i[b])
            pltpu.async_copy(d_hbm.at[pl.ds(fb, frag)],
                             d_v[b].at[pl.ds(0, frag)], sem_i[b])

        def wait_in(b):
            pltpu.make_async_copy(idx_hbm.at[pl.ds(0, frag)], idx_v[b],
                                  sem_i[b]).wait()
            pltpu.make_async_copy(d_hbm.at[pl.ds(0, frag)],
                                  d_v[b].at[pl.ds(0, frag)], sem_i[b]).wait()

        def issue_gather(b):
            for j in range(nstr):
                pltpu.async_copy(
                    feat_hbm.at[idx_v[b].at[pl.ds(j * _GB, _GB)]],
                    rows_v[b].at[pl.ds(j * _GB, _GB), :], sem_g[b])

        def wait_gather(b):
            for j in range(nstr):
                pltpu.make_async_copy(
                    feat_hbm.at[idx_v[b].at[pl.ds(j * _GB, _GB)]],
                    rows_v[b].at[pl.ds(j * _GB, _GB), :], sem_g[b]).wait()

        def compute(ci, b):
            dd, rr, oo = d_v[b], rows_v[b], out_v

            def _weights(pi):
                w = (jnp.float32(1.0)
                     - dd[pl.ds(pi * (2 * K), _L)] * jnp.float32(_INV_R2))
                t = w + _perm(w, ix1)
                t = t + _perm(t, ix2)
                dvec = (t + _perm(t, ix4)) + jnp.float32(1e-10)
                return w, jnp.float32(1.0) / dvec

            @pl.loop(0, _CH // 2, init_carry=_weights(0))
            def _pair(pi, carry):
                w, rcp = carry
                nxt = _weights(pi + 1)
                fpb = pi * (2 * K)
                p0 = pi * 2
                for px in range(2):
                    acc_lo = jnp.zeros((_L,), jnp.float32)
                    acc_hi = jnp.zeros((_L,), jnp.float32)
                    for kk in range(K):
                        wb = _bcast(w, px * K + kk)
                        f = fpb + px * K + kk
                        acc_lo = acc_lo + wb * rr[f, pl.ds(0, _L)]
                        acc_hi = acc_hi + wb * rr[f, pl.ds(_L, _L)]
                    rb = _bcast(rcp, px * K)
                    ob = (p0 + px) * C
                    oo[pl.ds(ob, _L)] = acc_lo * rb
                    oo[pl.ds(ob + _L, _L)] = acc_hi * rb
                return nxt

            pltpu.sync_copy(
                oo, out_hbm.at[pl.ds((pix_base + ci * _CH) * C, _CH * C)])

        issue_in(0, 0)
        issue_in(1, 1)
        wait_in(0)
        issue_gather(0)

        @pl.loop(0, nch // 2)
        def _steps(si):
            for b in range(2):
                ci = si * 2 + b
                wait_gather(b)
                nb = 1 - b

                @pl.when(ci + 1 < nch)
                def _():
                    wait_in(nb)
                    issue_gather(nb)

                compute(ci, b)

                @pl.when(ci + 2 < nch)
                def _():
                    issue_in(ci + 2, b)

    return k(idx_f, d_f, features)


def kernel(idx, dists, features):
    B, H, W, K = idx.shape
    P, C = features.shape
    n_pix = B * H * W
    assert n_pix % (_NW * _CH) == 0
    idx_f = idx.reshape(n_pix * K).astype(jnp.int32)
    d_f = dists.reshape(n_pix * K).astype(jnp.float32)
    out = _render(idx_f, d_f, features, n_pix=n_pix, k_frag=K, n_chan=C)
    return out.reshape(B, H, W, C)
